# baseline (speedup 1.0000x reference)
import functools

import jax
import jax.numpy as jnp
from jax import lax
from jax.experimental import pallas as pl
from jax.experimental.pallas import tpu as pltpu
from jax.experimental.pallas import tpu_sc as plsc

HID = 768
B = 4
S = 2048
MAXPOS = 2048
EPS = 1e-12
L = 16
NSL = HID // L
NC = 2
NS = 16
NW = NC * NS
TOK = B * S
TPW = TOK // NW
CH = 64
NCH = TPW // CH


def _scaledsin_table():
    pos = jnp.arange(MAXPOS, dtype=jnp.float32)
    half_d = HID // 2
    freq_seq = -jnp.arange(half_d, dtype=jnp.float32) / float(half_d)
    inv_freq = jnp.power(10000.0, freq_seq)
    sinusoid = pos[:, None] * inv_freq[None, :]
    return jnp.concatenate([jnp.sin(sinusoid), jnp.cos(sinusoid)], axis=-1)


def _make_sc_kernel():
    mesh = plsc.VectorSubcoreMesh(core_axis_name="c", subcore_axis_name="s")

    @functools.partial(
        pl.kernel,
        mesh=mesh,
        out_type=jax.ShapeDtypeStruct((TOK, HID), jnp.float32),
        scratch_types=[
            pltpu.VMEM((TPW,), jnp.int32),
            pltpu.VMEM((TPW + L,), jnp.int32),
            pltpu.VMEM((CH, HID), jnp.float32),
            pltpu.VMEM((CH, HID), jnp.float32),
            pltpu.VMEM((2, HID), jnp.float32),
            pltpu.VMEM((L,), jnp.float32),
            pltpu.VMEM((HID,), jnp.float32),
            pltpu.VMEM((HID,), jnp.float32),
            pltpu.SemaphoreType.DMA,
        ],
    )
    def emb_kernel(word_hbm, ids_hbm, tts_hbm, pos_hbm, type_hbm, sin_hbm,
                   gamma_hbm, beta_hbm, out_hbm,
                   idx_v, tts_v, wbuf, pbuf, tbuf, sinv, gv, bv,
                   sem):
        wid = lax.axis_index("s") * NC + lax.axis_index("c")
        base = wid * TPW
        s0 = base % S

        pltpu.sync_copy(ids_hbm.at[pl.ds(base, TPW)], idx_v)
        pltpu.sync_copy(tts_hbm.at[pl.ds(base, TPW)], tts_v.at[pl.ds(0, TPW)])
        pltpu.sync_copy(type_hbm, tbuf)
        pltpu.sync_copy(sin_hbm, sinv)
        pltpu.sync_copy(gamma_hbm, gv)
        pltpu.sync_copy(beta_hbm, bv)
        sv = sinv[...]

        for c in range(NCH):
            row0 = base + c * CH
            gather = pltpu.async_copy(
                word_hbm.at[idx_v.at[pl.ds(c * CH, CH)]], wbuf, sem)
            pltpu.sync_copy(pos_hbm.at[pl.ds(s0 + c * CH, CH)], pbuf)
            gather.wait()

            def per_token(t, _):
                tt = tts_v[pl.ds(c * CH + t, L)][0]
                acc = jnp.zeros((L,), jnp.float32)
                acc2 = jnp.zeros((L,), jnp.float32)
                for j in range(NSL):
                    sl = pl.ds(j * L, L)
                    e = wbuf[t, sl] + tbuf[tt, sl] + pbuf[t, sl] * sv
                    wbuf[t, sl] = e
                    acc = acc + e
                    acc2 = acc2 + e * e
                iota = lax.iota(jnp.int32, L)
                for d in (8, 4, 2, 1):
                    perm = iota ^ d
                    acc = acc + jnp.take(acc, perm, mode="wrap")
                    acc2 = acc2 + jnp.take(acc2, perm, mode="wrap")
                m16 = acc * (1.0 / HID)
                x = acc2 * (1.0 / HID) - m16 * m16 + EPS
                i = lax.bitcast_convert_type(x, jnp.int32)
                i = 0x5F3759DF - lax.shift_right_arithmetic(i, 1)
                y = lax.bitcast_convert_type(i, jnp.float32)
                y = y * (1.5 - 0.5 * x * y * y)
                y = y * (1.5 - 0.5 * x * y * y)
                r16 = y * (1.5 - 0.5 * x * y * y)
                for j in range(NSL):
                    sl = pl.ds(j * L, L)
                    wbuf[t, sl] = (wbuf[t, sl] - m16) * r16 * gv[sl] + bv[sl]
                return 0

            lax.fori_loop(0, CH, per_token, 0)

            pltpu.sync_copy(wbuf, out_hbm.at[pl.ds(row0, CH)])

    return emb_kernel


_sc_kernel = _make_sc_kernel()


def kernel(input_ids, token_type_ids, word_table, type_table, ln_gamma,
           ln_beta, sin_scalar):
    ids = input_ids.reshape(TOK).astype(jnp.int32)
    tts = token_type_ids.reshape(TOK).astype(jnp.int32)
    pos = _scaledsin_table()
    sinv = jnp.broadcast_to(sin_scalar.astype(jnp.float32).reshape(()), (L,))
    out = _sc_kernel(word_table.astype(jnp.float32), ids, tts, pos,
                     type_table.astype(jnp.float32), sinv,
                     ln_gamma.astype(jnp.float32), ln_beta.astype(jnp.float32))
    return out.reshape(B, S, HID)
